# Initial kernel scaffold; baseline (speedup 1.0000x reference)
import jax
import jax.numpy as jnp
from jax.experimental import pallas as pl


def kernel(subtask, results, agent_keys, router_w, router_b, merger_w, merger_b, norm_w):
    raise NotImplementedError("write your pallas kernel here")


# speedup vs baseline: 1.6437x; 1.6437x over previous
import functools

import jax
import jax.numpy as jnp
from jax import lax
from jax.experimental import pallas as pl
from jax.experimental.pallas import tpu as pltpu
from jax.experimental.pallas import tpu_sc as plsc

D_BRAIN = 1024
N_AGENTS = 8
TOK = 8192
BT = 1024


def _score_body(sub_ref, rw_ref, kt_ref, rb_ref, out_ref):
    x = sub_ref[...].astype(jnp.bfloat16)
    q = jnp.dot(x, rw_ref[...], preferred_element_type=jnp.float32) + rb_ref[...]
    out_ref[...] = jnp.dot(q.astype(jnp.bfloat16), kt_ref[...],
                           preferred_element_type=jnp.float32)


def _merge_body(sub_ref, res_ref, w_ref, mb_ref, nw_ref, out_ref):
    a = pl.program_id(1)
    x = res_ref[0].astype(jnp.bfloat16)
    acc = jnp.dot(x, w_ref[0], preferred_element_type=jnp.float32)

    @pl.when(a == 0)
    def _():
        out_ref[...] = acc

    @pl.when(a != 0)
    def _():
        out_ref[...] = out_ref[...] + acc

    @pl.when(a == N_AGENTS - 1)
    def _():
        m = out_ref[...] + sub_ref[...] + mb_ref[...]
        var = jnp.mean(m * m, axis=-1, keepdims=True)
        out_ref[...] = (m * lax.rsqrt(var + 1e-6)) * nw_ref[...]


@functools.lru_cache(maxsize=1)
def _make_top3_sc_kernel():
    mesh = plsc.VectorSubcoreMesh(core_axis_name="c", subcore_axis_name="s")

    @functools.partial(
        pl.kernel,
        mesh=mesh,
        out_type=jax.ShapeDtypeStruct((16,), jnp.int32),
        scratch_types=[
            pltpu.VMEM((16,), jnp.float32),
            pltpu.VMEM((16,), jnp.int32),
        ],
        compiler_params=pltpu.CompilerParams(needs_layout_passes=False),
    )
    def _top3_sc_kernel(s_hbm, out_hbm, svec, ovec):
        pltpu.sync_copy(s_hbm, svec)
        idx = lax.iota(jnp.int32, 16)
        _, order = plsc.sort_key_val(svec[...], idx, descending=True)
        ovec[...] = order

        @pl.when(jnp.logical_and(lax.axis_index("c") == 0,
                                 lax.axis_index("s") == 0))
        def _():
            pltpu.sync_copy(ovec, out_hbm)

    return _top3_sc_kernel


def kernel(subtask, results, agent_keys, router_w, router_b, merger_w,
           merger_b, norm_w):
    f32 = jnp.float32
    rw16 = router_w.astype(jnp.bfloat16)
    kt_pad = jnp.zeros((D_BRAIN, 128), jnp.bfloat16)
    kt_pad = kt_pad.at[:, :N_AGENTS].set(agent_keys.T.astype(jnp.bfloat16))
    rb = router_b.reshape(1, D_BRAIN)

    n_t = TOK // BT
    scores_pad = pl.pallas_call(
        _score_body,
        grid=(n_t,),
        in_specs=[
            pl.BlockSpec((BT, D_BRAIN), lambda t: (t, 0)),
            pl.BlockSpec((D_BRAIN, D_BRAIN), lambda t: (0, 0)),
            pl.BlockSpec((D_BRAIN, 128), lambda t: (0, 0)),
            pl.BlockSpec((1, D_BRAIN), lambda t: (0, 0)),
        ],
        out_specs=pl.BlockSpec((BT, 128), lambda t: (t, 0)),
        out_shape=jax.ShapeDtypeStruct((TOK, 128), f32),
        compiler_params=pltpu.CompilerParams(
            dimension_semantics=("parallel",)),
    )(subtask, rw16, kt_pad, rb)
    scores = scores_pad[:, :N_AGENTS]

    w3 = merger_w.astype(jnp.bfloat16).reshape(N_AGENTS, D_BRAIN, D_BRAIN)
    mb = merger_b.reshape(1, D_BRAIN)
    nw = norm_w.reshape(1, D_BRAIN)
    out = pl.pallas_call(
        _merge_body,
        grid=(n_t, N_AGENTS),
        in_specs=[
            pl.BlockSpec((BT, D_BRAIN), lambda t, a: (t, 0)),
            pl.BlockSpec((1, BT, D_BRAIN), lambda t, a: (a, t, 0)),
            pl.BlockSpec((1, D_BRAIN, D_BRAIN), lambda t, a: (a, 0, 0)),
            pl.BlockSpec((1, D_BRAIN), lambda t, a: (0, 0)),
            pl.BlockSpec((1, D_BRAIN), lambda t, a: (0, 0)),
        ],
        out_specs=pl.BlockSpec((BT, D_BRAIN), lambda t, a: (t, 0)),
        out_shape=jax.ShapeDtypeStruct((TOK, D_BRAIN), f32),
        compiler_params=pltpu.CompilerParams(
            dimension_semantics=("parallel", "arbitrary")),
    )(subtask, results, w3, mb, nw)

    svec_in = jnp.concatenate(
        [scores_pad[0, :N_AGENTS], jnp.full((8,), -jnp.inf, f32)])
    top3_16 = _make_top3_sc_kernel()(svec_in)
    top3 = top3_16[:3]
    return (out, scores, top3)


# speedup vs baseline: 1.9456x; 1.1836x over previous
import functools

import jax
import jax.numpy as jnp
from jax import lax
from jax.experimental import pallas as pl
from jax.experimental.pallas import tpu as pltpu
from jax.experimental.pallas import tpu_sc as plsc

D_BRAIN = 1024
N_AGENTS = 8
TOK = 8192
BT = 1024
BT_MERGE = 256


def _score_body(sub_ref, rw_ref, kt_ref, rb_ref, out_ref):
    x = sub_ref[...].astype(jnp.bfloat16)
    q = jnp.dot(x, rw_ref[...], preferred_element_type=jnp.float32) + rb_ref[...]
    out_ref[...] = jnp.dot(q.astype(jnp.bfloat16), kt_ref[...],
                           preferred_element_type=jnp.float32)


def _merge_body(sub_ref, res_ref, w_ref, mb_ref, nw_ref, out_ref):
    acc = sub_ref[...] + mb_ref[...]
    for a in range(N_AGENTS):
        x = res_ref[a].astype(jnp.bfloat16)
        acc = acc + jnp.dot(x, w_ref[a], preferred_element_type=jnp.float32)
    var = jnp.mean(acc * acc, axis=-1, keepdims=True)
    out_ref[...] = (acc * lax.rsqrt(var + 1e-6)) * nw_ref[...]


@functools.lru_cache(maxsize=1)
def _make_top3_sc_kernel():
    mesh = plsc.VectorSubcoreMesh(core_axis_name="c", subcore_axis_name="s")

    @functools.partial(
        pl.kernel,
        mesh=mesh,
        out_type=jax.ShapeDtypeStruct((16,), jnp.int32),
        scratch_types=[
            pltpu.VMEM((16,), jnp.float32),
            pltpu.VMEM((16,), jnp.int32),
        ],
        compiler_params=pltpu.CompilerParams(needs_layout_passes=False),
    )
    def _top3_sc_kernel(s_hbm, out_hbm, svec, ovec):
        pltpu.sync_copy(s_hbm, svec)
        idx = lax.iota(jnp.int32, 16)
        _, order = plsc.sort_key_val(svec[...], idx, descending=True)
        ovec[...] = order

        @pl.when(jnp.logical_and(lax.axis_index("c") == 0,
                                 lax.axis_index("s") == 0))
        def _():
            pltpu.sync_copy(ovec, out_hbm)

    return _top3_sc_kernel


def kernel(subtask, results, agent_keys, router_w, router_b, merger_w,
           merger_b, norm_w):
    f32 = jnp.float32
    rw16 = router_w.astype(jnp.bfloat16)
    kt_pad = jnp.zeros((D_BRAIN, 128), jnp.bfloat16)
    kt_pad = kt_pad.at[:, :N_AGENTS].set(agent_keys.T.astype(jnp.bfloat16))
    rb = router_b.reshape(1, D_BRAIN)

    n_t = TOK // BT
    scores_pad = pl.pallas_call(
        _score_body,
        grid=(n_t,),
        in_specs=[
            pl.BlockSpec((BT, D_BRAIN), lambda t: (t, 0)),
            pl.BlockSpec((D_BRAIN, D_BRAIN), lambda t: (0, 0)),
            pl.BlockSpec((D_BRAIN, 128), lambda t: (0, 0)),
            pl.BlockSpec((1, D_BRAIN), lambda t: (0, 0)),
        ],
        out_specs=pl.BlockSpec((BT, 128), lambda t: (t, 0)),
        out_shape=jax.ShapeDtypeStruct((TOK, 128), f32),
        compiler_params=pltpu.CompilerParams(
            dimension_semantics=("parallel",)),
    )(subtask, rw16, kt_pad, rb)
    scores = scores_pad[:, :N_AGENTS]

    w3 = merger_w.astype(jnp.bfloat16).reshape(N_AGENTS, D_BRAIN, D_BRAIN)
    mb = merger_b.reshape(1, D_BRAIN)
    nw = norm_w.reshape(1, D_BRAIN)
    n_tm = TOK // BT_MERGE
    out = pl.pallas_call(
        _merge_body,
        grid=(n_tm,),
        in_specs=[
            pl.BlockSpec((BT_MERGE, D_BRAIN), lambda t: (t, 0)),
            pl.BlockSpec((N_AGENTS, BT_MERGE, D_BRAIN), lambda t: (0, t, 0)),
            pl.BlockSpec((N_AGENTS, D_BRAIN, D_BRAIN), lambda t: (0, 0, 0)),
            pl.BlockSpec((1, D_BRAIN), lambda t: (0, 0)),
            pl.BlockSpec((1, D_BRAIN), lambda t: (0, 0)),
        ],
        out_specs=pl.BlockSpec((BT_MERGE, D_BRAIN), lambda t: (t, 0)),
        out_shape=jax.ShapeDtypeStruct((TOK, D_BRAIN), f32),
        compiler_params=pltpu.CompilerParams(
            dimension_semantics=("parallel",)),
    )(subtask, results, w3, mb, nw)

    svec_in = jnp.concatenate(
        [scores_pad[0, :N_AGENTS], jnp.full((8,), -jnp.inf, f32)])
    top3_16 = _make_top3_sc_kernel()(svec_in)
    top3 = top3_16[:3]
    return (out, scores, top3)


# speedup vs baseline: 1.9835x; 1.0195x over previous
import functools

import jax
import jax.numpy as jnp
from jax import lax
from jax.experimental import pallas as pl
from jax.experimental.pallas import tpu as pltpu
from jax.experimental.pallas import tpu_sc as plsc

D_BRAIN = 1024
N_AGENTS = 8
TOK = 8192
BT = 1024
BT_MERGE = 256


def _fused_body(sub_ref, res_ref, w_ref, rw_ref, kt_ref, rb_ref, mb_ref,
                nw_ref, out_ref, sc_ref):
    x = sub_ref[...]
    xb = x.astype(jnp.bfloat16)
    q = jnp.dot(xb, rw_ref[...], preferred_element_type=jnp.float32)
    q = q + rb_ref[...]
    sc_ref[...] = jnp.dot(q.astype(jnp.bfloat16), kt_ref[...],
                          preferred_element_type=jnp.float32)
    acc = x + mb_ref[...]
    for a in range(N_AGENTS):
        xa = res_ref[a].astype(jnp.bfloat16)
        acc = acc + jnp.dot(xa, w_ref[a], preferred_element_type=jnp.float32)
    var = jnp.mean(acc * acc, axis=-1, keepdims=True)
    out_ref[...] = (acc * lax.rsqrt(var + 1e-6)) * nw_ref[...]


@functools.lru_cache(maxsize=1)
def _make_top3_sc_kernel():
    mesh = plsc.VectorSubcoreMesh(core_axis_name="c", subcore_axis_name="s")

    @functools.partial(
        pl.kernel,
        mesh=mesh,
        out_type=jax.ShapeDtypeStruct((16,), jnp.int32),
        scratch_types=[
            pltpu.VMEM((16,), jnp.float32),
            pltpu.VMEM((16,), jnp.int32),
        ],
        compiler_params=pltpu.CompilerParams(needs_layout_passes=False),
    )
    def _top3_sc_kernel(s_hbm, out_hbm, svec, ovec):
        pltpu.sync_copy(s_hbm, svec)
        idx = lax.iota(jnp.int32, 16)
        _, order = plsc.sort_key_val(svec[...], idx, descending=True)
        ovec[...] = order

        @pl.when(jnp.logical_and(lax.axis_index("c") == 0,
                                 lax.axis_index("s") == 0))
        def _():
            pltpu.sync_copy(ovec, out_hbm)

    return _top3_sc_kernel


def kernel(subtask, results, agent_keys, router_w, router_b, merger_w,
           merger_b, norm_w):
    f32 = jnp.float32
    rw16 = router_w.astype(jnp.bfloat16)
    kt_pad = jnp.zeros((D_BRAIN, 128), jnp.bfloat16)
    kt_pad = kt_pad.at[:, :N_AGENTS].set(agent_keys.T.astype(jnp.bfloat16))
    rb = router_b.reshape(1, D_BRAIN)

    w3 = merger_w.astype(jnp.bfloat16).reshape(N_AGENTS, D_BRAIN, D_BRAIN)
    mb = merger_b.reshape(1, D_BRAIN)
    nw = norm_w.reshape(1, D_BRAIN)
    n_tm = TOK // BT_MERGE
    out, scores_pad = pl.pallas_call(
        _fused_body,
        grid=(n_tm,),
        in_specs=[
            pl.BlockSpec((BT_MERGE, D_BRAIN), lambda t: (t, 0)),
            pl.BlockSpec((N_AGENTS, BT_MERGE, D_BRAIN), lambda t: (0, t, 0)),
            pl.BlockSpec((N_AGENTS, D_BRAIN, D_BRAIN), lambda t: (0, 0, 0)),
            pl.BlockSpec((D_BRAIN, D_BRAIN), lambda t: (0, 0)),
            pl.BlockSpec((D_BRAIN, 128), lambda t: (0, 0)),
            pl.BlockSpec((1, D_BRAIN), lambda t: (0, 0)),
            pl.BlockSpec((1, D_BRAIN), lambda t: (0, 0)),
            pl.BlockSpec((1, D_BRAIN), lambda t: (0, 0)),
        ],
        out_specs=[
            pl.BlockSpec((BT_MERGE, D_BRAIN), lambda t: (t, 0)),
            pl.BlockSpec((BT_MERGE, 128), lambda t: (t, 0)),
        ],
        out_shape=[
            jax.ShapeDtypeStruct((TOK, D_BRAIN), f32),
            jax.ShapeDtypeStruct((TOK, 128), f32),
        ],
        compiler_params=pltpu.CompilerParams(
            dimension_semantics=("parallel",)),
    )(subtask, results, w3, rw16, kt_pad, rb, mb, nw)
    scores = scores_pad[:, :N_AGENTS]

    svec_in = jnp.concatenate(
        [scores_pad[0, :N_AGENTS], jnp.full((8,), -jnp.inf, f32)])
    top3_16 = _make_top3_sc_kernel()(svec_in)
    top3 = top3_16[:3]
    return (out, scores, top3)


# speedup vs baseline: 2.0109x; 1.0138x over previous
import functools

import jax
import jax.numpy as jnp
from jax import lax
from jax.experimental import pallas as pl
from jax.experimental.pallas import tpu as pltpu
from jax.experimental.pallas import tpu_sc as plsc

D_BRAIN = 1024
N_AGENTS = 8
TOK = 8192
BT = 1024
BT_MERGE = 512


def _fused_body(sub_ref, res_ref, w_ref, rw_ref, kt_ref, rb_ref, mb_ref,
                nw_ref, out_ref, sc_ref):
    x = sub_ref[...]
    xb = x.astype(jnp.bfloat16)
    q = jnp.dot(xb, rw_ref[...], preferred_element_type=jnp.float32)
    q = q + rb_ref[...]
    sc_ref[...] = jnp.dot(q.astype(jnp.bfloat16), kt_ref[...],
                          preferred_element_type=jnp.float32)
    acc = x + mb_ref[...]
    for a in range(N_AGENTS):
        xa = res_ref[a].astype(jnp.bfloat16)
        acc = acc + jnp.dot(xa, w_ref[a], preferred_element_type=jnp.float32)
    var = jnp.mean(acc * acc, axis=-1, keepdims=True)
    out_ref[...] = (acc * lax.rsqrt(var + 1e-6)) * nw_ref[...]


@functools.lru_cache(maxsize=1)
def _make_top3_sc_kernel():
    mesh = plsc.VectorSubcoreMesh(core_axis_name="c", subcore_axis_name="s")

    @functools.partial(
        pl.kernel,
        mesh=mesh,
        out_type=jax.ShapeDtypeStruct((16,), jnp.int32),
        scratch_types=[
            pltpu.VMEM((16,), jnp.float32),
            pltpu.VMEM((16,), jnp.int32),
        ],
        compiler_params=pltpu.CompilerParams(needs_layout_passes=False),
    )
    def _top3_sc_kernel(s_hbm, out_hbm, svec, ovec):
        pltpu.sync_copy(s_hbm, svec)
        idx = lax.iota(jnp.int32, 16)
        _, order = plsc.sort_key_val(svec[...], idx, descending=True)
        ovec[...] = order

        @pl.when(jnp.logical_and(lax.axis_index("c") == 0,
                                 lax.axis_index("s") == 0))
        def _():
            pltpu.sync_copy(ovec, out_hbm)

    return _top3_sc_kernel


def kernel(subtask, results, agent_keys, router_w, router_b, merger_w,
           merger_b, norm_w):
    f32 = jnp.float32
    rw16 = router_w.astype(jnp.bfloat16)
    kt_pad = jnp.zeros((D_BRAIN, 128), jnp.bfloat16)
    kt_pad = kt_pad.at[:, :N_AGENTS].set(agent_keys.T.astype(jnp.bfloat16))
    rb = router_b.reshape(1, D_BRAIN)

    w3 = merger_w.astype(jnp.bfloat16).reshape(N_AGENTS, D_BRAIN, D_BRAIN)
    mb = merger_b.reshape(1, D_BRAIN)
    nw = norm_w.reshape(1, D_BRAIN)
    n_tm = TOK // BT_MERGE
    out, scores_pad = pl.pallas_call(
        _fused_body,
        grid=(n_tm,),
        in_specs=[
            pl.BlockSpec((BT_MERGE, D_BRAIN), lambda t: (t, 0)),
            pl.BlockSpec((N_AGENTS, BT_MERGE, D_BRAIN), lambda t: (0, t, 0)),
            pl.BlockSpec((N_AGENTS, D_BRAIN, D_BRAIN), lambda t: (0, 0, 0)),
            pl.BlockSpec((D_BRAIN, D_BRAIN), lambda t: (0, 0)),
            pl.BlockSpec((D_BRAIN, 128), lambda t: (0, 0)),
            pl.BlockSpec((1, D_BRAIN), lambda t: (0, 0)),
            pl.BlockSpec((1, D_BRAIN), lambda t: (0, 0)),
            pl.BlockSpec((1, D_BRAIN), lambda t: (0, 0)),
        ],
        out_specs=[
            pl.BlockSpec((BT_MERGE, D_BRAIN), lambda t: (t, 0)),
            pl.BlockSpec((BT_MERGE, 128), lambda t: (t, 0)),
        ],
        out_shape=[
            jax.ShapeDtypeStruct((TOK, D_BRAIN), f32),
            jax.ShapeDtypeStruct((TOK, 128), f32),
        ],
        compiler_params=pltpu.CompilerParams(
            dimension_semantics=("parallel",),
            vmem_limit_bytes=100 * 1024 * 1024),
    )(subtask, results, w3, rw16, kt_pad, rb, mb, nw)
    scores = scores_pad[:, :N_AGENTS]

    svec_in = jnp.concatenate(
        [scores_pad[0, :N_AGENTS], jnp.full((8,), -jnp.inf, f32)])
    top3_16 = _make_top3_sc_kernel()(svec_in)
    top3 = top3_16[:3]
    return (out, scores, top3)
